# Initial kernel scaffold; baseline (speedup 1.0000x reference)
import functools

import jax
import jax.numpy as jnp
from jax import lax
from jax.experimental import pallas as pl
from jax.experimental.pallas import tpu as pltpu
from jax.experimental.pallas import tpu_sc as plsc

N = 10000
E = 320000
D = 128
NC = 2
NS = 16
NW = NC * NS
CHUNK = 128
CPT = 79
EPAD = NW * CPT * CHUNK
NPAD = 10016
RPT = NPAD // NS
OPT = N // NS
ZCOPIES = (RPT + CHUNK - 1) // CHUNK

_BLK = 2000


def _agg_call(h, src_p, dst_p):
    mesh = plsc.VectorSubcoreMesh(core_axis_name="c", subcore_axis_name="s")

    @functools.partial(
        pl.kernel,
        mesh=mesh,
        out_type=jax.ShapeDtypeStruct((NC, N, D), jnp.float32),
        scratch_types=[
            pltpu.VMEM((CPT, CHUNK), jnp.int32),
            pltpu.VMEM((CPT, CHUNK), jnp.int32),
            pltpu.VMEM((CHUNK, D), jnp.float32),
            pltpu.VMEM((CHUNK, D), jnp.float32),
            pltpu.VMEM_SHARED((NPAD, D), jnp.float32),
            pltpu.SemaphoreType.DMA,
        ],
    )
    def agg_kernel(h_hbm, src_hbm, dst_hbm, out_hbm,
                   src_v, dst_v, rows_v, zbuf, agg_sh, sem):
        c = lax.axis_index("c")
        s = lax.axis_index("s")
        wid = c * NS + s

        pltpu.sync_copy(src_hbm.at[wid], src_v)
        pltpu.sync_copy(dst_hbm.at[wid], dst_v)

        zero = jnp.zeros((16,), jnp.float32)

        def zrow(i, _):
            for l in range(D // 16):
                zbuf[i, pl.ds(l * 16, 16)] = zero
            return ()

        lax.fori_loop(0, CHUNK, zrow, ())

        zbase = s * RPT
        for k in range(ZCOPIES):
            nrows = min(CHUNK, RPT - k * CHUNK)
            pltpu.sync_copy(zbuf.at[pl.ds(0, nrows)],
                            agg_sh.at[pl.ds(zbase + k * CHUNK, nrows)])
        plsc.subcore_barrier()

        def body(j, _):
            pltpu.async_copy(h_hbm.at[src_v.at[j]], rows_v, sem).wait()
            pltpu.sync_copy(rows_v, agg_sh.at[dst_v.at[j]], add=True)
            return ()

        lax.fori_loop(0, CPT, body, ())
        plsc.subcore_barrier()

        obase = s * OPT
        pltpu.sync_copy(agg_sh.at[pl.ds(obase, OPT)],
                        out_hbm.at[c, pl.ds(obase, OPT)])

    return agg_kernel(h, src_p, dst_p)


def _lin_in(x, W, b):
    def body(x_ref, w_ref, b_ref, o_ref):
        o_ref[...] = jnp.tanh(
            jnp.dot(x_ref[...], w_ref[...], preferred_element_type=jnp.float32)
            + b_ref[...])

    return pl.pallas_call(
        body,
        grid=(N // _BLK,),
        in_specs=[
            pl.BlockSpec((_BLK, D), lambda i: (i, 0)),
            pl.BlockSpec((D, D), lambda i: (0, 0)),
            pl.BlockSpec((1, D), lambda i: (0, 0)),
        ],
        out_specs=pl.BlockSpec((_BLK, D), lambda i: (i, 0)),
        out_shape=jax.ShapeDtypeStruct((N, D), jnp.float32),
    )(x, W, b.reshape(1, D))


def _layer(agg2, h, Wr, Ws, b):
    def body(a_ref, h_ref, wr_ref, ws_ref, b_ref, o_ref):
        a = a_ref[0] + a_ref[1]
        o_ref[...] = jnp.tanh(
            jnp.dot(a, wr_ref[...], preferred_element_type=jnp.float32)
            + jnp.dot(h_ref[...], ws_ref[...], preferred_element_type=jnp.float32)
            + b_ref[...])

    return pl.pallas_call(
        body,
        grid=(N // _BLK,),
        in_specs=[
            pl.BlockSpec((NC, _BLK, D), lambda i: (0, i, 0)),
            pl.BlockSpec((_BLK, D), lambda i: (i, 0)),
            pl.BlockSpec((D, D), lambda i: (0, 0)),
            pl.BlockSpec((D, D), lambda i: (0, 0)),
            pl.BlockSpec((1, D), lambda i: (0, 0)),
        ],
        out_specs=pl.BlockSpec((_BLK, D), lambda i: (i, 0)),
        out_shape=jax.ShapeDtypeStruct((N, D), jnp.float32),
    )(agg2, h, Wr, Ws, b.reshape(1, D))


def _lin_out(x, W, b):
    def body(x_ref, w_ref, b_ref, o_ref):
        o_ref[...] = jnp.maximum(
            jnp.dot(x_ref[...], w_ref[...], preferred_element_type=jnp.float32)
            + b_ref[...], 0.0)

    return pl.pallas_call(
        body,
        grid=(N // _BLK,),
        in_specs=[
            pl.BlockSpec((_BLK, D), lambda i: (i, 0)),
            pl.BlockSpec((D, D), lambda i: (0, 0)),
            pl.BlockSpec((1, D), lambda i: (0, 0)),
        ],
        out_specs=pl.BlockSpec((_BLK, D), lambda i: (i, 0)),
        out_shape=jax.ShapeDtypeStruct((N, D), jnp.float32),
    )(x, W, b.reshape(1, D))


def kernel(x, edge_index, lin1_W, lin1_b,
           g1_Wr, g1_Ws, g1_b,
           g2_Wr, g2_Ws, g2_b,
           g3_Wr, g3_Ws, g3_b,
           g4_Wr, g4_Ws, g4_b,
           lin2_W, lin2_b):
    pad = EPAD - E
    src_p = jnp.concatenate(
        [edge_index[0], jnp.zeros((pad,), jnp.int32)]).reshape(NW, CPT, CHUNK)
    dst_p = jnp.concatenate(
        [edge_index[1], jnp.full((pad,), N, jnp.int32)]).reshape(NW, CPT, CHUNK)

    h = _lin_in(x, lin1_W, lin1_b)
    for Wr, Ws, b in ((g1_Wr, g1_Ws, g1_b), (g2_Wr, g2_Ws, g2_b),
                      (g3_Wr, g3_Ws, g3_b), (g4_Wr, g4_Ws, g4_b)):
        agg2 = _agg_call(h, src_p, dst_p)
        h = _layer(agg2, h, Wr, Ws, b)
    return _lin_out(h, lin2_W, lin2_b)


# speedup vs baseline: 5.8336x; 5.8336x over previous
import functools

import jax
import jax.numpy as jnp
from jax import lax
from jax.experimental import pallas as pl
from jax.experimental.pallas import tpu as pltpu
from jax.experimental.pallas import tpu_sc as plsc

N = 10000
E = 320000
D = 128
H = 64
NC = 2
NS = 16
CHUNK = 128
EPT = E // NS
CPT = (EPT + CHUNK - 1) // CHUNK
EPT_PAD = CPT * CHUNK
NPAD = 10016
RPT = NPAD // NS
OPT = 624
ZCOPIES = (RPT + CHUNK - 1) // CHUNK

_BLK = 2000


def _agg_call(h2, src_p, dst_p):
    mesh = plsc.VectorSubcoreMesh(core_axis_name="c", subcore_axis_name="s")

    @functools.partial(
        pl.kernel,
        mesh=mesh,
        compiler_params=pltpu.CompilerParams(use_tc_tiling_on_sc=False),
        out_type=jax.ShapeDtypeStruct((NC, N, H), jnp.float32),
        scratch_types=[
            pltpu.VMEM((CPT, CHUNK), jnp.int32),
            pltpu.VMEM((CPT, CHUNK), jnp.int32),
            pltpu.VMEM((CHUNK, H), jnp.float32),
            pltpu.VMEM((CHUNK, H), jnp.float32),
            pltpu.VMEM_SHARED((NPAD, H), jnp.float32),
            pltpu.SemaphoreType.DMA,
        ],
    )
    def agg_kernel(h_hbm, src_hbm, dst_hbm, out_hbm,
                   src_v, dst_v, rows_v, zbuf, agg_sh, sem):
        c = lax.axis_index("c")
        s = lax.axis_index("s")

        pltpu.sync_copy(src_hbm.at[s], src_v)
        pltpu.sync_copy(dst_hbm.at[s], dst_v)

        zero = jnp.zeros((16,), jnp.float32)

        def zrow(i, _):
            for l in range(H // 16):
                zbuf[i, pl.ds(l * 16, 16)] = zero
            return ()

        lax.fori_loop(0, CHUNK, zrow, ())

        zbase = s * RPT
        for k in range(ZCOPIES):
            nrows = min(CHUNK, RPT - k * CHUNK)
            pltpu.sync_copy(zbuf.at[pl.ds(0, nrows)],
                            agg_sh.at[pl.ds(zbase + k * CHUNK, nrows)])
        plsc.subcore_barrier()

        def body(j, _):
            pltpu.async_copy(h_hbm.at[c].at[src_v.at[j]], rows_v, sem).wait()
            pltpu.sync_copy(rows_v, agg_sh.at[dst_v.at[j]], add=True)
            return ()

        lax.fori_loop(0, CPT, body, ())
        plsc.subcore_barrier()

        obase = s * OPT
        pltpu.sync_copy(agg_sh.at[pl.ds(obase, OPT)],
                        out_hbm.at[c, pl.ds(obase, OPT)])

        @pl.when(s == NS - 1)
        def _tail():
            pltpu.sync_copy(agg_sh.at[pl.ds(NS * OPT, N - NS * OPT)],
                            out_hbm.at[c, pl.ds(NS * OPT, N - NS * OPT)])

    return agg_kernel(h2, src_p, dst_p)


def _split(o_ref, res):
    o_ref[0] = res[:, :H]
    o_ref[1] = res[:, H:]


def _lin_in(x, W, b):
    def body(x_ref, w_ref, b_ref, o_ref):
        res = jnp.tanh(
            jnp.dot(x_ref[...], w_ref[...], preferred_element_type=jnp.float32)
            + b_ref[...])
        _split(o_ref, res)

    return pl.pallas_call(
        body,
        grid=(N // _BLK,),
        in_specs=[
            pl.BlockSpec((_BLK, D), lambda i: (i, 0)),
            pl.BlockSpec((D, D), lambda i: (0, 0)),
            pl.BlockSpec((1, D), lambda i: (0, 0)),
        ],
        out_specs=pl.BlockSpec((NC, _BLK, H), lambda i: (0, i, 0)),
        out_shape=jax.ShapeDtypeStruct((NC, N, H), jnp.float32),
    )(x, W, b.reshape(1, D))


def _layer(agg2, h2, Wr, Ws, b):
    def body(a_ref, h_ref, wr_ref, ws_ref, b_ref, o_ref):
        a = jnp.concatenate([a_ref[0], a_ref[1]], axis=1)
        h = jnp.concatenate([h_ref[0], h_ref[1]], axis=1)
        res = jnp.tanh(
            jnp.dot(a, wr_ref[...], preferred_element_type=jnp.float32)
            + jnp.dot(h, ws_ref[...], preferred_element_type=jnp.float32)
            + b_ref[...])
        _split(o_ref, res)

    return pl.pallas_call(
        body,
        grid=(N // _BLK,),
        in_specs=[
            pl.BlockSpec((NC, _BLK, H), lambda i: (0, i, 0)),
            pl.BlockSpec((NC, _BLK, H), lambda i: (0, i, 0)),
            pl.BlockSpec((D, D), lambda i: (0, 0)),
            pl.BlockSpec((D, D), lambda i: (0, 0)),
            pl.BlockSpec((1, D), lambda i: (0, 0)),
        ],
        out_specs=pl.BlockSpec((NC, _BLK, H), lambda i: (0, i, 0)),
        out_shape=jax.ShapeDtypeStruct((NC, N, H), jnp.float32),
    )(agg2, h2, Wr, Ws, b.reshape(1, D))


def _lin_out(h2, W, b):
    def body(h_ref, w_ref, b_ref, o_ref):
        h = jnp.concatenate([h_ref[0], h_ref[1]], axis=1)
        o_ref[...] = jnp.maximum(
            jnp.dot(h, w_ref[...], preferred_element_type=jnp.float32)
            + b_ref[...], 0.0)

    return pl.pallas_call(
        body,
        grid=(N // _BLK,),
        in_specs=[
            pl.BlockSpec((NC, _BLK, H), lambda i: (0, i, 0)),
            pl.BlockSpec((D, D), lambda i: (0, 0)),
            pl.BlockSpec((1, D), lambda i: (0, 0)),
        ],
        out_specs=pl.BlockSpec((_BLK, D), lambda i: (i, 0)),
        out_shape=jax.ShapeDtypeStruct((N, D), jnp.float32),
    )(h2, W, b.reshape(1, D))


def kernel(x, edge_index, lin1_W, lin1_b,
           g1_Wr, g1_Ws, g1_b,
           g2_Wr, g2_Ws, g2_b,
           g3_Wr, g3_Ws, g3_b,
           g4_Wr, g4_Ws, g4_b,
           lin2_W, lin2_b):
    pad = EPT_PAD - EPT
    src_p = jnp.pad(edge_index[0].reshape(NS, EPT),
                    ((0, 0), (0, pad))).reshape(NS, CPT, CHUNK)
    dst_p = jnp.pad(edge_index[1].reshape(NS, EPT),
                    ((0, 0), (0, pad)),
                    constant_values=N).reshape(NS, CPT, CHUNK)

    h2 = _lin_in(x, lin1_W, lin1_b)
    for Wr, Ws, b in ((g1_Wr, g1_Ws, g1_b), (g2_Wr, g2_Ws, g2_b),
                      (g3_Wr, g3_Ws, g3_b), (g4_Wr, g4_Ws, g4_b)):
        agg2 = _agg_call(h2, src_p, dst_p)
        h2 = _layer(agg2, h2, Wr, Ws, b)
    return _lin_out(h2, lin2_W, lin2_b)
